# baseline (speedup 1.0000x reference)
import jax
import jax.numpy as jnp
from jax import lax
from jax.experimental import pallas as pl
from jax.experimental.pallas import tpu as pltpu
from jax.experimental.pallas import tpu_sc as plsc

L = 16384
NTAGS = 16
NUM_SUBCORES = 16
CHUNK = 128
CHUNKS_PER_WORKER = L // (NUM_SUBCORES * CHUNK)


def _bow_body(words_hbm, table_hbm, bias_hbm, out_hbm,
              idx_v, buf0, buf1, acc_v, shared, tmp_v, bias_v, sem0, sem1):
    wid = lax.axis_index("s")

    pltpu.sync_copy(words_hbm.at[wid], idx_v)

    bufs = (buf0, buf1)
    sems = (sem0, sem1)
    copies = [None, None]
    copies[0] = pltpu.async_copy(table_hbm.at[idx_v.at[0]], bufs[0], sems[0])

    acc = jnp.zeros((NTAGS,), jnp.float32)
    for c in range(CHUNKS_PER_WORKER):
        if c + 1 < CHUNKS_PER_WORKER:
            nb = (c + 1) % 2
            copies[nb] = pltpu.async_copy(
                table_hbm.at[idx_v.at[c + 1]], bufs[nb], sems[nb])
        copies[c % 2].wait()
        buf = bufs[c % 2]

        def inner(j, a, buf=buf):
            return a + buf[j, :]
        acc = lax.fori_loop(0, CHUNK, inner, acc)

    acc_v[...] = acc
    pltpu.sync_copy(acc_v, shared.at[wid])
    plsc.subcore_barrier()

    @pl.when(wid == 0)
    def _():
        pltpu.sync_copy(shared, tmp_v)
        pltpu.sync_copy(bias_hbm, bias_v)
        tot = bias_v[...]
        for j in range(NUM_SUBCORES):
            tot = tot + tmp_v[j, :]
        acc_v[...] = tot
        pltpu.sync_copy(acc_v, out_hbm.at[0])


def kernel(words, embedding, bias):
    words3d = words.astype(jnp.int32).reshape(
        NUM_SUBCORES, CHUNKS_PER_WORKER, CHUNK)
    mesh = plsc.VectorSubcoreMesh(
        core_axis_name="c", subcore_axis_name="s", num_cores=1)
    k = pl.kernel(
        _bow_body,
        out_type=jax.ShapeDtypeStruct((1, NTAGS), jnp.float32),
        mesh=mesh,
        scratch_types=[
            pltpu.VMEM((CHUNKS_PER_WORKER, CHUNK), jnp.int32),
            pltpu.VMEM((CHUNK, NTAGS), jnp.float32),
            pltpu.VMEM((CHUNK, NTAGS), jnp.float32),
            pltpu.VMEM((NTAGS,), jnp.float32),
            pltpu.VMEM_SHARED((NUM_SUBCORES, NTAGS), jnp.float32),
            pltpu.VMEM((NUM_SUBCORES, NTAGS), jnp.float32),
            pltpu.VMEM((NTAGS,), jnp.float32),
            pltpu.SemaphoreType.DMA,
            pltpu.SemaphoreType.DMA,
        ],
        compiler_params=pltpu.CompilerParams(use_tc_tiling_on_sc=False),
    )
    return k(words3d, embedding, bias)


# speedup vs baseline: 1.6501x; 1.6501x over previous
import jax
import jax.numpy as jnp
from jax import lax
from jax.experimental import pallas as pl
from jax.experimental.pallas import tpu as pltpu
from jax.experimental.pallas import tpu_sc as plsc

L = 16384
NTAGS = 16
NUM_SUBCORES = 16
VECL = 16
ROWS_PER_WORKER = L // NUM_SUBCORES
ROUND = 256
VECS_PER_ROUND = ROUND // VECL
NROUNDS = ROWS_PER_WORKER // ROUND


def _fire_round(table_hbm, idx_v, r, buf, sem):
    def f(g, _):
        iv = idx_v[r * VECS_PER_ROUND + g, :]
        for k in range(VECL):
            pltpu.async_copy(table_hbm.at[iv[k]], buf.at[g * VECL + k], sem)
        return 0
    lax.fori_loop(0, VECS_PER_ROUND, f, 0)


def _drain_acc_round(table_hbm, buf, sem, acc):
    def d(j, acc):
        pltpu.make_async_copy(table_hbm.at[0], buf.at[j], sem).wait()
        return acc + buf[j, :]
    return lax.fori_loop(0, ROUND, d, acc)


def _bow_body(words_hbm, table_hbm, bias_hbm, out_hbm, partials_hbm,
              idx_v, buf_a, buf_b, acc_v, tmp_v, bias_v,
              sem_a, sem_b):
    wid = lax.axis_index("s")

    pltpu.sync_copy(words_hbm.at[wid], idx_v)

    acc = jnp.zeros((NTAGS,), jnp.float32)
    _fire_round(table_hbm, idx_v, 0, buf_a, sem_a)
    _fire_round(table_hbm, idx_v, 1, buf_b, sem_b)
    acc = _drain_acc_round(table_hbm, buf_a, sem_a, acc)
    _fire_round(table_hbm, idx_v, 2, buf_a, sem_a)
    acc = _drain_acc_round(table_hbm, buf_b, sem_b, acc)
    _fire_round(table_hbm, idx_v, 3, buf_b, sem_b)
    acc = _drain_acc_round(table_hbm, buf_a, sem_a, acc)
    acc = _drain_acc_round(table_hbm, buf_b, sem_b, acc)

    acc_v[...] = acc
    pltpu.sync_copy(acc_v, partials_hbm.at[wid])
    plsc.subcore_barrier()

    @pl.when(wid == 0)
    def _():
        pltpu.sync_copy(partials_hbm, tmp_v)
        pltpu.sync_copy(bias_hbm, bias_v)
        tot = bias_v[...]
        for j in range(NUM_SUBCORES):
            tot = tot + tmp_v[j, :]
        acc_v[...] = tot
        pltpu.sync_copy(acc_v, out_hbm.at[0])


def kernel(words, embedding, bias):
    words3d = words.astype(jnp.int32).reshape(
        NUM_SUBCORES, ROWS_PER_WORKER // VECL, VECL)
    mesh = plsc.VectorSubcoreMesh(
        core_axis_name="c", subcore_axis_name="s", num_cores=1)
    k = pl.kernel(
        _bow_body,
        out_type=(jax.ShapeDtypeStruct((1, NTAGS), jnp.float32),
                  jax.ShapeDtypeStruct((NUM_SUBCORES, NTAGS), jnp.float32)),
        mesh=mesh,
        scratch_types=[
            pltpu.VMEM((ROWS_PER_WORKER // VECL, VECL), jnp.int32),
            pltpu.VMEM((ROUND, NTAGS), jnp.float32),
            pltpu.VMEM((ROUND, NTAGS), jnp.float32),
            pltpu.VMEM((NTAGS,), jnp.float32),
            pltpu.VMEM((NUM_SUBCORES, NTAGS), jnp.float32),
            pltpu.VMEM((NTAGS,), jnp.float32),
            pltpu.SemaphoreType.DMA,
            pltpu.SemaphoreType.DMA,
        ],
        compiler_params=pltpu.CompilerParams(use_tc_tiling_on_sc=True),
    )
    out, _ = k(words3d, embedding, bias)
    return out
